# baseline (speedup 1.0000x reference)
import functools

import jax
import jax.numpy as jnp
from jax import lax
from jax.experimental import pallas as pl
from jax.experimental.pallas import tpu as pltpu
from jax.experimental.pallas import tpu_sc as plsc

N = 10000
E = 320000
D = 128
HD = 64
NP = 10240
NW = 32
B = 64
CHUNKS = 160
EP = NW * CHUNKS * B
STRIPE = NP // 16

_f32 = jnp.float32


def _dot_t(x, w):
    return lax.dot_general(x, w, (((1,), (1,)), ((), ())),
                           preferred_element_type=jnp.float32)


def _pre_body(x_ref, w_ref, b_ref, a_ref, g_ref, e_ref, h_ref, f_ref):
    h = _dot_t(x_ref[...], w_ref[...]) + b_ref[...]
    asrc = _dot_t(h, a_ref[:, :D])
    adst = _dot_t(h, a_ref[:, D:])
    e = jnp.exp(asrc - jnp.max(asrc, axis=0, keepdims=True))
    f_ref[...] = jnp.exp(adst - jnp.max(adst, axis=0, keepdims=True))
    h64 = h[:, :HD]
    g_ref[pl.ds(0, N), :] = jnp.concatenate(
        [h64 * e[:, 0:1], h64 * e[:, 1:2]], axis=1)
    g_ref[pl.ds(N, NP - N), :] = jnp.zeros((NP - N, D), _f32)
    e_ref[pl.ds(0, N), :] = e
    e_ref[pl.ds(N, NP - N), :] = jnp.zeros((NP - N, 2), _f32)
    h_ref[...] = h


def _finish_layer(p0, p1, t0, t1, h, f):
    acc = p0[:N] + p1[:N]
    t = t0[:N] + t1[:N]
    z = jnp.sum(f * t, axis=0, keepdims=True)
    w = f / z
    agg = jnp.concatenate(
        [acc[:, :HD] * w[:, 0:1], acc[:, HD:] * w[:, 1:2]], axis=1)
    return jax.nn.relu(agg + h)


def _fin_body(p0_ref, p1_ref, t0_ref, t1_ref, h_ref, f_ref, x_ref):
    x_ref[...] = _finish_layer(p0_ref[...], p1_ref[...], t0_ref[...],
                               t1_ref[...], h_ref[...], f_ref[...])


def _post_body(p0_ref, p1_ref, t0_ref, t1_ref, h_ref, f_ref,
               wc1_ref, bc1_ref, wc2_ref, bc2_ref, out_ref):
    x2 = _finish_layer(p0_ref[...], p1_ref[...], t0_ref[...], t1_ref[...],
                       h_ref[...], f_ref[...])
    hc = jax.nn.relu(_dot_t(x2, wc1_ref[...]) + bc1_ref[...])
    out_ref[...] = _dot_t(hc, wc2_ref[...]) + bc2_ref[...]


_pre_call = pl.pallas_call(
    _pre_body,
    out_shape=(jax.ShapeDtypeStruct((NP, D), _f32),
               jax.ShapeDtypeStruct((NP, 2), _f32),
               jax.ShapeDtypeStruct((N, D), _f32),
               jax.ShapeDtypeStruct((N, 2), _f32)))

_fin_call = pl.pallas_call(
    _fin_body,
    out_shape=jax.ShapeDtypeStruct((N, D), _f32))

_post_call = pl.pallas_call(
    _post_body,
    out_shape=jax.ShapeDtypeStruct((N, 2), _f32))


def _sc_body(g_hbm, e_hbm, src_hbm, dst_hbm, zg_hbm, ze_hbm,
             out_hbm, outt_hbm,
             src_v, dst_v, rows_v, trows_v, acc_sh, acct_sh, sem, semt):
    c = lax.axis_index("c")
    s = lax.axis_index("s")
    wid = c * 16 + s
    pltpu.sync_copy(src_hbm.at[wid], src_v)
    pltpu.sync_copy(dst_hbm.at[wid], dst_v)
    pltpu.sync_copy(zg_hbm, acc_sh.at[pl.ds(s * STRIPE, STRIPE)])
    pltpu.sync_copy(ze_hbm, acct_sh.at[pl.ds(s * STRIPE, STRIPE)])
    plsc.subcore_barrier()

    pltpu.async_copy(g_hbm.at[src_v.at[0]], rows_v.at[0], sem)
    pltpu.async_copy(e_hbm.at[src_v.at[0]], trows_v.at[0], semt)

    def body(i, _):
        slot = lax.rem(i, 2)
        nxt = lax.rem(i + 1, 2)

        @pl.when(i + 1 < CHUNKS)
        def _prefetch():
            pltpu.async_copy(g_hbm.at[src_v.at[i + 1]], rows_v.at[nxt], sem)
            pltpu.async_copy(e_hbm.at[src_v.at[i + 1]], trows_v.at[nxt], semt)

        pltpu.make_async_copy(g_hbm.at[src_v.at[i]], rows_v.at[slot],
                              sem).wait()
        pltpu.make_async_copy(e_hbm.at[src_v.at[i]], trows_v.at[slot],
                              semt).wait()
        pltpu.sync_copy(rows_v.at[slot], acc_sh.at[dst_v.at[i]], add=True)
        pltpu.sync_copy(trows_v.at[slot], acct_sh.at[dst_v.at[i]], add=True)
        return 0

    lax.fori_loop(0, CHUNKS, body, 0)
    plsc.subcore_barrier()
    pltpu.sync_copy(acc_sh.at[pl.ds(s * STRIPE, STRIPE)],
                    out_hbm.at[c, pl.ds(s * STRIPE, STRIPE)])
    pltpu.sync_copy(acct_sh.at[pl.ds(s * STRIPE, STRIPE)],
                    outt_hbm.at[c, pl.ds(s * STRIPE, STRIPE)])


@functools.cache
def _sc_segsum_call():
    mesh = plsc.VectorSubcoreMesh(core_axis_name="c", subcore_axis_name="s",
                                  num_cores=2, num_subcores=16)
    return pl.kernel(
        _sc_body,
        out_type=(jax.ShapeDtypeStruct((2, NP, D), _f32),
                  jax.ShapeDtypeStruct((2, NP, 2), _f32)),
        mesh=mesh,
        compiler_params=pltpu.CompilerParams(use_tc_tiling_on_sc=False),
        scratch_types=[
            pltpu.VMEM((CHUNKS, B), jnp.int32),
            pltpu.VMEM((CHUNKS, B), jnp.int32),
            pltpu.VMEM((2, B, D), _f32),
            pltpu.VMEM((2, B, 2), _f32),
            pltpu.VMEM_SHARED((NP, D), _f32),
            pltpu.VMEM_SHARED((NP, 2), _f32),
            pltpu.SemaphoreType.DMA,
            pltpu.SemaphoreType.DMA,
        ])


def kernel(features, edge_indices, edge_weights, W0, b0, A0, ab0,
           W1, b1, A1, ab1, Wp, bp, Wc1, bc1, Wc2, bc2):
    del edge_weights, ab0, ab1, Wp, bp
    edge_index = edge_indices[0]
    pad = jnp.full((EP - E,), N, jnp.int32)
    src3 = jnp.concatenate([edge_index[0], pad]).reshape(NW, CHUNKS, B)
    dst3 = jnp.concatenate([edge_index[1], pad]).reshape(NW, CHUNKS, B)
    zg = jnp.zeros((STRIPE, D), _f32)
    ze = jnp.zeros((STRIPE, 2), _f32)

    sc = _sc_segsum_call()
    g0, e0, h0, f0 = _pre_call(features, W0, b0.reshape(1, D), A0)
    p, pt = sc(g0, e0, src3, dst3, zg, ze)
    x1 = _fin_call(p[0], p[1], pt[0], pt[1], h0, f0)
    g1, e1, h1, f1 = _pre_call(x1, W1, b1.reshape(1, D), A1)
    q, qt = sc(g1, e1, src3, dst3, zg, ze)
    logits = _post_call(q[0], q[1], qt[0], qt[1], h1, f1,
                        Wc1, bc1.reshape(1, HD), Wc2, bc2.reshape(1, 2))
    return logits


# speedup vs baseline: 2.5663x; 2.5663x over previous
import functools

import jax
import jax.numpy as jnp
from jax import lax
from jax.experimental import pallas as pl
from jax.experimental.pallas import tpu as pltpu
from jax.experimental.pallas import tpu_sc as plsc

N = 10000
E = 320000
D = 128
HD = 64
NP = 10240
NW = 32
B = 64
CHUNKS = 160
EP = NW * CHUNKS * B
STRIPE = NP // 16

_f32 = jnp.float32


def _dot_t(x, w):
    return lax.dot_general(x, w, (((1,), (1,)), ((), ())),
                           preferred_element_type=jnp.float32)


def _pre_body(x_ref, w_ref, b_ref, a_ref, g_ref, e_ref, h_ref, f_ref):
    h = _dot_t(x_ref[...], w_ref[...]) + b_ref[...]
    asrc = _dot_t(h, a_ref[:, :D])
    adst = _dot_t(h, a_ref[:, D:])
    e = jnp.exp(asrc - jnp.max(asrc, axis=0, keepdims=True))
    f_ref[...] = jnp.exp(adst - jnp.max(adst, axis=0, keepdims=True))
    h64 = h[:, :HD]
    g_ref[pl.ds(0, N), :] = jnp.concatenate(
        [h64 * e[:, 0:1], h64 * e[:, 1:2]], axis=1)
    g_ref[pl.ds(N, NP - N), :] = jnp.zeros((NP - N, D), _f32)
    e_ref[pl.ds(0, N), :] = e
    e_ref[pl.ds(N, NP - N), :] = jnp.zeros((NP - N, 2), _f32)
    h_ref[...] = h


def _finish_layer(p0, p1, t0, t1, h, f):
    acc = p0[:N] + p1[:N]
    t = t0[:N] + t1[:N]
    z = jnp.sum(f * t, axis=0, keepdims=True)
    w = f / z
    agg = jnp.concatenate(
        [acc[:, :HD] * w[:, 0:1], acc[:, HD:] * w[:, 1:2]], axis=1)
    return jax.nn.relu(agg + h)


def _fin_body(p0_ref, p1_ref, t0_ref, t1_ref, h_ref, f_ref, x_ref):
    x_ref[...] = _finish_layer(p0_ref[...], p1_ref[...], t0_ref[...],
                               t1_ref[...], h_ref[...], f_ref[...])


def _post_body(p0_ref, p1_ref, t0_ref, t1_ref, h_ref, f_ref,
               wc1_ref, bc1_ref, wc2_ref, bc2_ref, out_ref):
    x2 = _finish_layer(p0_ref[...], p1_ref[...], t0_ref[...], t1_ref[...],
                       h_ref[...], f_ref[...])
    hc = jax.nn.relu(_dot_t(x2, wc1_ref[...]) + bc1_ref[...])
    out_ref[...] = _dot_t(hc, wc2_ref[...]) + bc2_ref[...]


_pre_call = pl.pallas_call(
    _pre_body,
    out_shape=(jax.ShapeDtypeStruct((NP, D), _f32),
               jax.ShapeDtypeStruct((NP, 2), _f32),
               jax.ShapeDtypeStruct((N, D), _f32),
               jax.ShapeDtypeStruct((N, 2), _f32)))

_fin_call = pl.pallas_call(
    _fin_body,
    out_shape=jax.ShapeDtypeStruct((N, D), _f32))

_post_call = pl.pallas_call(
    _post_body,
    out_shape=jax.ShapeDtypeStruct((N, 2), _f32))


def _sc_body(g_hbm, e_hbm, src_hbm, dst_hbm, zg_hbm, ze_hbm,
             out_hbm, outt_hbm,
             src_v, dst_v, rows_v, trows_v, acc_sh, acct_sh, sem, semt):
    c = lax.axis_index("c")
    s = lax.axis_index("s")
    wid = c * 16 + s
    pltpu.sync_copy(src_hbm.at[wid], src_v)
    pltpu.sync_copy(dst_hbm.at[wid], dst_v)
    pltpu.sync_copy(zg_hbm, acc_sh.at[pl.ds(s * STRIPE, STRIPE)])
    pltpu.sync_copy(ze_hbm, acct_sh.at[pl.ds(s * STRIPE, STRIPE)])
    plsc.subcore_barrier()

    pltpu.async_copy(g_hbm.at[src_v.at[0]], rows_v.at[0], sem)
    pltpu.async_copy(e_hbm.at[src_v.at[0]], trows_v.at[0], semt)

    def body(i, _):
        slot = lax.rem(i, 2)
        nxt = lax.rem(i + 1, 2)

        @pl.when(i + 1 < CHUNKS)
        def _prefetch():
            pltpu.async_copy(g_hbm.at[src_v.at[i + 1]], rows_v.at[nxt], sem)
            pltpu.async_copy(e_hbm.at[src_v.at[i + 1]], trows_v.at[nxt], semt)

        pltpu.make_async_copy(g_hbm.at[src_v.at[i]], rows_v.at[slot],
                              sem).wait()
        pltpu.make_async_copy(e_hbm.at[src_v.at[i]], trows_v.at[slot],
                              semt).wait()
        pltpu.sync_copy(rows_v.at[slot], acc_sh.at[dst_v.at[i]], add=True)
        pltpu.sync_copy(trows_v.at[slot], acct_sh.at[dst_v.at[i]], add=True)
        return 0

    lax.fori_loop(0, CHUNKS, body, 0)
    plsc.subcore_barrier()
    pltpu.sync_copy(acc_sh.at[pl.ds(s * STRIPE, STRIPE)],
                    out_hbm.at[c, pl.ds(s * STRIPE, STRIPE)])
    pltpu.sync_copy(acct_sh.at[pl.ds(s * STRIPE, STRIPE)],
                    outt_hbm.at[c, pl.ds(s * STRIPE, STRIPE)])


@functools.cache
def _sc_segsum_call():
    mesh = plsc.VectorSubcoreMesh(core_axis_name="c", subcore_axis_name="s",
                                  num_cores=2, num_subcores=16)
    return pl.kernel(
        _sc_body,
        out_type=(jax.ShapeDtypeStruct((2, NP, D), _f32),
                  jax.ShapeDtypeStruct((2, NP, 2), _f32)),
        mesh=mesh,
        compiler_params=pltpu.CompilerParams(use_tc_tiling_on_sc=False),
        scratch_types=[
            pltpu.VMEM((CHUNKS, B), jnp.int32),
            pltpu.VMEM((CHUNKS, B), jnp.int32),
            pltpu.VMEM((2, B, D), _f32),
            pltpu.VMEM((2, B, 2), _f32),
            pltpu.VMEM_SHARED((NP, D), _f32),
            pltpu.VMEM_SHARED((NP, 2), _f32),
            pltpu.SemaphoreType.DMA,
            pltpu.SemaphoreType.DMA,
        ])


def kernel(features, edge_indices, edge_weights, W0, b0, A0, ab0,
           W1, b1, A1, ab1, Wp, bp, Wc1, bc1, Wc2, bc2):
    del edge_weights, ab0, ab1, Wp, bp
    edge_index = edge_indices[0]
    pad = N + jnp.arange(EP - E, dtype=jnp.int32) % (NP - N)
    src3 = jnp.concatenate([edge_index[0], pad]).reshape(NW, CHUNKS, B)
    dst3 = jnp.concatenate([edge_index[1], pad]).reshape(NW, CHUNKS, B)
    zg = jnp.zeros((STRIPE, D), _f32)
    ze = jnp.zeros((STRIPE, 2), _f32)

    sc = _sc_segsum_call()
    g0, e0, h0, f0 = _pre_call(features, W0, b0.reshape(1, D), A0)
    p, pt = sc(g0, e0, src3, dst3, zg, ze)
    x1 = _fin_call(p[0], p[1], pt[0], pt[1], h0, f0)
    g1, e1, h1, f1 = _pre_call(x1, W1, b1.reshape(1, D), A1)
    q, qt = sc(g1, e1, src3, dst3, zg, ze)
    logits = _post_call(q[0], q[1], qt[0], qt[1], h1, f1,
                        Wc1, bc1.reshape(1, HD), Wc2, bc2.reshape(1, 2))
    return logits


# speedup vs baseline: 2.7065x; 1.0546x over previous
import functools

import jax
import jax.numpy as jnp
from jax import lax
from jax.experimental import pallas as pl
from jax.experimental.pallas import tpu as pltpu
from jax.experimental.pallas import tpu_sc as plsc

N = 10000
E = 320000
D = 128
HD = 64
NP = 10240
NW = 32
B = 64
CHUNKS = 160
EP = NW * CHUNKS * B
STRIPE = NP // 16

_f32 = jnp.float32


def _dot_t(x, w):
    return lax.dot_general(x, w, (((1,), (1,)), ((), ())),
                           preferred_element_type=jnp.float32)


def _pre_body(x_ref, w_ref, b_ref, a_ref, g_ref, e_ref, h_ref, f_ref):
    h = _dot_t(x_ref[...], w_ref[...]) + b_ref[...]
    asrc = _dot_t(h, a_ref[:, :D])
    adst = _dot_t(h, a_ref[:, D:])
    e = jnp.exp(asrc - jnp.max(asrc, axis=0, keepdims=True))
    f_ref[...] = jnp.exp(adst - jnp.max(adst, axis=0, keepdims=True))
    h64 = h[:, :HD]
    g_ref[pl.ds(0, N), :] = jnp.concatenate(
        [h64 * e[:, 0:1], h64 * e[:, 1:2]], axis=1)
    g_ref[pl.ds(N, NP - N), :] = jnp.zeros((NP - N, D), _f32)
    e_ref[pl.ds(0, N), :] = e
    e_ref[pl.ds(N, NP - N), :] = jnp.zeros((NP - N, 2), _f32)
    h_ref[...] = h


def _finish_layer(p, t2, h, f):
    acc = p[0, :N] + p[1, :N]
    t = t2[0, :N] + t2[1, :N]
    z = jnp.sum(f * t, axis=0, keepdims=True)
    w = f / z
    agg = jnp.concatenate(
        [acc[:, :HD] * w[:, 0:1], acc[:, HD:] * w[:, 1:2]], axis=1)
    return jax.nn.relu(agg + h)


def _fin_body(p_ref, t_ref, h_ref, f_ref, x_ref):
    x_ref[...] = _finish_layer(p_ref[...], t_ref[...], h_ref[...], f_ref[...])


def _post_body(p_ref, t_ref, h_ref, f_ref,
               wc1_ref, bc1_ref, wc2_ref, bc2_ref, out_ref):
    x2 = _finish_layer(p_ref[...], t_ref[...], h_ref[...], f_ref[...])
    hc = jax.nn.relu(_dot_t(x2, wc1_ref[...]) + bc1_ref[...])
    out_ref[...] = _dot_t(hc, wc2_ref[...]) + bc2_ref[...]


_pre_call = pl.pallas_call(
    _pre_body,
    out_shape=(jax.ShapeDtypeStruct((NP, D), _f32),
               jax.ShapeDtypeStruct((NP, 2), _f32),
               jax.ShapeDtypeStruct((N, D), _f32),
               jax.ShapeDtypeStruct((N, 2), _f32)))

_fin_call = pl.pallas_call(
    _fin_body,
    out_shape=jax.ShapeDtypeStruct((N, D), _f32))

_post_call = pl.pallas_call(
    _post_body,
    out_shape=jax.ShapeDtypeStruct((N, 2), _f32))


def _sc_body(g_hbm, e_hbm, src_hbm, dst_hbm, zg_hbm, ze_hbm,
             out_hbm, outt_hbm,
             src_v, dst_v, rows_v, trows_v, acc_sh, acct_sh, sem, semt):
    c = lax.axis_index("c")
    s = lax.axis_index("s")
    wid = c * 16 + s
    pltpu.sync_copy(src_hbm.at[wid], src_v)
    pltpu.sync_copy(dst_hbm.at[wid], dst_v)
    pltpu.sync_copy(zg_hbm, acc_sh.at[pl.ds(s * STRIPE, STRIPE)])
    pltpu.sync_copy(ze_hbm, acct_sh.at[pl.ds(s * STRIPE, STRIPE)])
    plsc.subcore_barrier()

    pltpu.async_copy(g_hbm.at[src_v.at[0]], rows_v.at[0], sem)
    pltpu.async_copy(e_hbm.at[src_v.at[0]], trows_v.at[0], semt)

    def body(i, _):
        slot = lax.rem(i, 2)
        nxt = lax.rem(i + 1, 2)

        @pl.when(i + 1 < CHUNKS)
        def _prefetch():
            pltpu.async_copy(g_hbm.at[src_v.at[i + 1]], rows_v.at[nxt], sem)
            pltpu.async_copy(e_hbm.at[src_v.at[i + 1]], trows_v.at[nxt], semt)

        pltpu.make_async_copy(g_hbm.at[src_v.at[i]], rows_v.at[slot],
                              sem).wait()
        pltpu.make_async_copy(e_hbm.at[src_v.at[i]], trows_v.at[slot],
                              semt).wait()
        pltpu.sync_copy(rows_v.at[slot], acc_sh.at[dst_v.at[i]], add=True)
        pltpu.sync_copy(trows_v.at[slot], acct_sh.at[dst_v.at[i]], add=True)
        return 0

    lax.fori_loop(0, CHUNKS, body, 0)
    plsc.subcore_barrier()
    pltpu.sync_copy(acc_sh.at[pl.ds(s * STRIPE, STRIPE)],
                    out_hbm.at[c, pl.ds(s * STRIPE, STRIPE)])
    pltpu.sync_copy(acct_sh.at[pl.ds(s * STRIPE, STRIPE)],
                    outt_hbm.at[c, pl.ds(s * STRIPE, STRIPE)])


@functools.cache
def _sc_segsum_call():
    mesh = plsc.VectorSubcoreMesh(core_axis_name="c", subcore_axis_name="s",
                                  num_cores=2, num_subcores=16)
    return pl.kernel(
        _sc_body,
        out_type=(jax.ShapeDtypeStruct((2, NP, D), _f32),
                  jax.ShapeDtypeStruct((2, NP, 2), _f32)),
        mesh=mesh,
        compiler_params=pltpu.CompilerParams(use_tc_tiling_on_sc=False),
        scratch_types=[
            pltpu.VMEM((CHUNKS, B), jnp.int32),
            pltpu.VMEM((CHUNKS, B), jnp.int32),
            pltpu.VMEM((2, B, D), _f32),
            pltpu.VMEM((2, B, 2), _f32),
            pltpu.VMEM_SHARED((NP, D), _f32),
            pltpu.VMEM_SHARED((NP, 2), _f32),
            pltpu.SemaphoreType.DMA,
            pltpu.SemaphoreType.DMA,
        ])


def kernel(features, edge_indices, edge_weights, W0, b0, A0, ab0,
           W1, b1, A1, ab1, Wp, bp, Wc1, bc1, Wc2, bc2):
    del edge_weights, ab0, ab1, Wp, bp
    edge_index = edge_indices[0]
    pad = N + jnp.arange(EP - E, dtype=jnp.int32) % (NP - N)
    src3 = jnp.concatenate([edge_index[0], pad]).reshape(NW, CHUNKS, B)
    dst3 = jnp.concatenate([edge_index[1], pad]).reshape(NW, CHUNKS, B)
    zg = jnp.zeros((STRIPE, D), _f32)
    ze = jnp.zeros((STRIPE, 2), _f32)

    sc = _sc_segsum_call()
    g0, e0, h0, f0 = _pre_call(features, W0, b0.reshape(1, D), A0)
    p, pt = sc(g0, e0, src3, dst3, zg, ze)
    x1 = _fin_call(p, pt, h0, f0)
    g1, e1, h1, f1 = _pre_call(x1, W1, b1.reshape(1, D), A1)
    q, qt = sc(g1, e1, src3, dst3, zg, ze)
    logits = _post_call(q, qt, h1, f1,
                        Wc1, bc1.reshape(1, HD), Wc2, bc2.reshape(1, 2))
    return logits


# speedup vs baseline: 2.8480x; 1.0523x over previous
import functools

import jax
import jax.numpy as jnp
from jax import lax
from jax.experimental import pallas as pl
from jax.experimental.pallas import tpu as pltpu
from jax.experimental.pallas import tpu_sc as plsc

N = 10000
E = 320000
D = 128
HD = 64
NP = 10240
NW = 32
B = 64
CHUNKS = 160
EP = NW * CHUNKS * B
STRIPE = NP // 16

_f32 = jnp.float32


def _dot_t(x, w):
    return lax.dot_general(x, w, (((1,), (1,)), ((), ())),
                           preferred_element_type=jnp.float32)


def _pre_body(x_ref, w_ref, b_ref, a_ref, g_ref, e_ref, h_ref, f_ref):
    h = _dot_t(x_ref[...], w_ref[...]) + b_ref[...]
    asrc = _dot_t(h, a_ref[:, :D])
    adst = _dot_t(h, a_ref[:, D:])
    e = jnp.exp(asrc - jnp.max(asrc, axis=0, keepdims=True))
    f_ref[...] = jnp.exp(adst - jnp.max(adst, axis=0, keepdims=True))
    h64 = h[:, :HD]
    g_ref[pl.ds(0, N), :] = jnp.concatenate(
        [h64 * e[:, 0:1], h64 * e[:, 1:2]], axis=1)
    g_ref[pl.ds(N, NP - N), :] = jnp.zeros((NP - N, D), _f32)
    e_ref[pl.ds(0, N), :] = e
    e_ref[pl.ds(N, NP - N), :] = jnp.zeros((NP - N, 2), _f32)
    h_ref[...] = h


def _finish_layer(p, t2, h, f):
    acc = p[0, :N] + p[1, :N]
    t = t2[0, :N] + t2[1, :N]
    z = jnp.sum(f * t, axis=0, keepdims=True)
    w = f / z
    agg = jnp.concatenate(
        [acc[:, :HD] * w[:, 0:1], acc[:, HD:] * w[:, 1:2]], axis=1)
    return jax.nn.relu(agg + h)


def _fin_body(p_ref, t_ref, h_ref, f_ref, x_ref):
    x_ref[...] = _finish_layer(p_ref[...], t_ref[...], h_ref[...], f_ref[...])


def _post_body(p_ref, t_ref, h_ref, f_ref,
               wc1_ref, bc1_ref, wc2_ref, bc2_ref, out_ref):
    x2 = _finish_layer(p_ref[...], t_ref[...], h_ref[...], f_ref[...])
    hc = jax.nn.relu(_dot_t(x2, wc1_ref[...]) + bc1_ref[...])
    out_ref[...] = _dot_t(hc, wc2_ref[...]) + bc2_ref[...]


_pre_call = pl.pallas_call(
    _pre_body,
    out_shape=(jax.ShapeDtypeStruct((NP, D), _f32),
               jax.ShapeDtypeStruct((NP, 2), _f32),
               jax.ShapeDtypeStruct((N, D), _f32),
               jax.ShapeDtypeStruct((N, 2), _f32)))

_fin_call = pl.pallas_call(
    _fin_body,
    out_shape=jax.ShapeDtypeStruct((N, D), _f32))

_post_call = pl.pallas_call(
    _post_body,
    out_shape=jax.ShapeDtypeStruct((N, 2), _f32))


def _sc_body(g_hbm, e_hbm, src_hbm, dst_hbm, zg_hbm, ze_hbm,
             out_hbm, outt_hbm,
             src_v, dst_v, rows_v, trows_v, acc_sh, acct_sh,
             semg0, semg1, semt0, semt1):
    semg = (semg0, semg1)
    semt = (semt0, semt1)
    c = lax.axis_index("c")
    s = lax.axis_index("s")
    wid = c * 16 + s
    pltpu.sync_copy(src_hbm.at[wid], src_v)
    pltpu.sync_copy(dst_hbm.at[wid], dst_v)
    pltpu.sync_copy(zg_hbm, acc_sh.at[pl.ds(s * STRIPE, STRIPE)])
    pltpu.sync_copy(ze_hbm, acct_sh.at[pl.ds(s * STRIPE, STRIPE)])
    plsc.subcore_barrier()

    for j in range(2):
        pltpu.async_copy(g_hbm.at[src_v.at[j]], rows_v.at[j], semg[j])
        pltpu.async_copy(e_hbm.at[src_v.at[j]], trows_v.at[j], semt[j])

    def body(k, _):
        for j in range(2):
            i = k * 2 + j
            pltpu.make_async_copy(g_hbm.at[src_v.at[i]], rows_v.at[j],
                                  semg[j]).wait()
            pltpu.make_async_copy(e_hbm.at[src_v.at[i]], trows_v.at[j],
                                  semt[j]).wait()
            pltpu.sync_copy(rows_v.at[j], acc_sh.at[dst_v.at[i]], add=True)
            pltpu.sync_copy(trows_v.at[j], acct_sh.at[dst_v.at[i]], add=True)

            @pl.when(i + 2 < CHUNKS)
            def _prefetch():
                pltpu.async_copy(g_hbm.at[src_v.at[i + 2]], rows_v.at[j],
                                 semg[j])
                pltpu.async_copy(e_hbm.at[src_v.at[i + 2]], trows_v.at[j],
                                 semt[j])
        return 0

    lax.fori_loop(0, CHUNKS // 2, body, 0)
    plsc.subcore_barrier()
    pltpu.sync_copy(acc_sh.at[pl.ds(s * STRIPE, STRIPE)],
                    out_hbm.at[c, pl.ds(s * STRIPE, STRIPE)])
    pltpu.sync_copy(acct_sh.at[pl.ds(s * STRIPE, STRIPE)],
                    outt_hbm.at[c, pl.ds(s * STRIPE, STRIPE)])


@functools.cache
def _sc_segsum_call():
    mesh = plsc.VectorSubcoreMesh(core_axis_name="c", subcore_axis_name="s",
                                  num_cores=2, num_subcores=16)
    return pl.kernel(
        _sc_body,
        out_type=(jax.ShapeDtypeStruct((2, NP, D), _f32),
                  jax.ShapeDtypeStruct((2, NP, 2), _f32)),
        mesh=mesh,
        compiler_params=pltpu.CompilerParams(use_tc_tiling_on_sc=False),
        scratch_types=[
            pltpu.VMEM((CHUNKS, B), jnp.int32),
            pltpu.VMEM((CHUNKS, B), jnp.int32),
            pltpu.VMEM((2, B, D), _f32),
            pltpu.VMEM((2, B, 2), _f32),
            pltpu.VMEM_SHARED((NP, D), _f32),
            pltpu.VMEM_SHARED((NP, 2), _f32),
            pltpu.SemaphoreType.DMA,
            pltpu.SemaphoreType.DMA,
            pltpu.SemaphoreType.DMA,
            pltpu.SemaphoreType.DMA,
        ])


def kernel(features, edge_indices, edge_weights, W0, b0, A0, ab0,
           W1, b1, A1, ab1, Wp, bp, Wc1, bc1, Wc2, bc2):
    del edge_weights, ab0, ab1, Wp, bp
    edge_index = edge_indices[0]
    pad = N + jnp.arange(EP - E, dtype=jnp.int32) % (NP - N)
    src3 = jnp.concatenate([edge_index[0], pad]).reshape(NW, CHUNKS, B)
    dst3 = jnp.concatenate([edge_index[1], pad]).reshape(NW, CHUNKS, B)
    zg = jnp.zeros((STRIPE, D), _f32)
    ze = jnp.zeros((STRIPE, 2), _f32)

    sc = _sc_segsum_call()
    g0, e0, h0, f0 = _pre_call(features, W0, b0.reshape(1, D), A0)
    p, pt = sc(g0, e0, src3, dst3, zg, ze)
    x1 = _fin_call(p, pt, h0, f0)
    g1, e1, h1, f1 = _pre_call(x1, W1, b1.reshape(1, D), A1)
    q, qt = sc(g1, e1, src3, dst3, zg, ze)
    logits = _post_call(q, qt, h1, f1,
                        Wc1, bc1.reshape(1, HD), Wc2, bc2.reshape(1, 2))
    return logits


# speedup vs baseline: 3.0093x; 1.0566x over previous
import functools

import jax
import jax.numpy as jnp
from jax import lax
from jax.experimental import pallas as pl
from jax.experimental.pallas import tpu as pltpu
from jax.experimental.pallas import tpu_sc as plsc

N = 10000
E = 320000
D = 128
HD = 64
NP = 10240
NW = 32
B = 80
CHUNKS = 128
EP = NW * CHUNKS * B
STRIPE = NP // 16

_f32 = jnp.float32


def _dot_t(x, w):
    return lax.dot_general(x, w, (((1,), (1,)), ((), ())),
                           preferred_element_type=jnp.float32)


def _pre_body(x_ref, w_ref, b_ref, a_ref, g_ref, e_ref, h_ref, f_ref):
    h = _dot_t(x_ref[...], w_ref[...]) + b_ref[...]
    asrc = _dot_t(h, a_ref[:, :D])
    adst = _dot_t(h, a_ref[:, D:])
    e = jnp.exp(asrc - jnp.max(asrc, axis=0, keepdims=True))
    f_ref[...] = jnp.exp(adst - jnp.max(adst, axis=0, keepdims=True))
    h64 = h[:, :HD]
    g_ref[pl.ds(0, N), :] = jnp.concatenate(
        [h64 * e[:, 0:1], h64 * e[:, 1:2]], axis=1)
    g_ref[pl.ds(N, NP - N), :] = jnp.zeros((NP - N, D), _f32)
    e_ref[pl.ds(0, N), :] = e
    e_ref[pl.ds(N, NP - N), :] = jnp.zeros((NP - N, 2), _f32)
    h_ref[...] = h


def _finish_layer(p, t2, h, f):
    acc = p[0, :N] + p[1, :N]
    t = t2[0, :N] + t2[1, :N]
    z = jnp.sum(f * t, axis=0, keepdims=True)
    w = f / z
    agg = jnp.concatenate(
        [acc[:, :HD] * w[:, 0:1], acc[:, HD:] * w[:, 1:2]], axis=1)
    return jax.nn.relu(agg + h)


def _fin_body(p_ref, t_ref, h_ref, f_ref, x_ref):
    x_ref[...] = _finish_layer(p_ref[...], t_ref[...], h_ref[...], f_ref[...])


def _post_body(p_ref, t_ref, h_ref, f_ref,
               wc1_ref, bc1_ref, wc2_ref, bc2_ref, out_ref):
    x2 = _finish_layer(p_ref[...], t_ref[...], h_ref[...], f_ref[...])
    hc = jax.nn.relu(_dot_t(x2, wc1_ref[...]) + bc1_ref[...])
    out_ref[...] = _dot_t(hc, wc2_ref[...]) + bc2_ref[...]


_pre_call = pl.pallas_call(
    _pre_body,
    out_shape=(jax.ShapeDtypeStruct((NP, D), _f32),
               jax.ShapeDtypeStruct((NP, 2), _f32),
               jax.ShapeDtypeStruct((N, D), _f32),
               jax.ShapeDtypeStruct((N, 2), _f32)))

_fin_call = pl.pallas_call(
    _fin_body,
    out_shape=jax.ShapeDtypeStruct((N, D), _f32))

_post_call = pl.pallas_call(
    _post_body,
    out_shape=jax.ShapeDtypeStruct((N, 2), _f32))


def _sc_body(g_hbm, e_hbm, src_hbm, dst_hbm, zg_hbm, ze_hbm,
             out_hbm, outt_hbm,
             src_v, dst_v, rows_v, trows_v, acc_sh, acct_sh,
             semg0, semg1, semt0, semt1):
    semg = (semg0, semg1)
    semt = (semt0, semt1)
    c = lax.axis_index("c")
    s = lax.axis_index("s")
    wid = c * 16 + s
    pltpu.sync_copy(src_hbm.at[wid], src_v)
    pltpu.sync_copy(dst_hbm.at[wid], dst_v)
    pltpu.sync_copy(zg_hbm, acc_sh.at[pl.ds(s * STRIPE, STRIPE)])
    pltpu.sync_copy(ze_hbm, acct_sh.at[pl.ds(s * STRIPE, STRIPE)])
    plsc.subcore_barrier()

    for j in range(2):
        pltpu.async_copy(g_hbm.at[src_v.at[j]], rows_v.at[j], semg[j])
        pltpu.async_copy(e_hbm.at[src_v.at[j]], trows_v.at[j], semt[j])

    def body(k, _):
        for j in range(2):
            i = k * 2 + j
            pltpu.make_async_copy(g_hbm.at[src_v.at[i]], rows_v.at[j],
                                  semg[j]).wait()
            pltpu.make_async_copy(e_hbm.at[src_v.at[i]], trows_v.at[j],
                                  semt[j]).wait()
            pltpu.sync_copy(rows_v.at[j], acc_sh.at[dst_v.at[i]], add=True)
            pltpu.sync_copy(trows_v.at[j], acct_sh.at[dst_v.at[i]], add=True)

            @pl.when(i + 2 < CHUNKS)
            def _prefetch():
                pltpu.async_copy(g_hbm.at[src_v.at[i + 2]], rows_v.at[j],
                                 semg[j])
                pltpu.async_copy(e_hbm.at[src_v.at[i + 2]], trows_v.at[j],
                                 semt[j])
        return 0

    lax.fori_loop(0, CHUNKS // 2, body, 0)
    plsc.subcore_barrier()
    pltpu.sync_copy(acc_sh.at[pl.ds(s * STRIPE, STRIPE)],
                    out_hbm.at[c, pl.ds(s * STRIPE, STRIPE)])
    pltpu.sync_copy(acct_sh.at[pl.ds(s * STRIPE, STRIPE)],
                    outt_hbm.at[c, pl.ds(s * STRIPE, STRIPE)])


@functools.cache
def _sc_segsum_call():
    mesh = plsc.VectorSubcoreMesh(core_axis_name="c", subcore_axis_name="s",
                                  num_cores=2, num_subcores=16)
    return pl.kernel(
        _sc_body,
        out_type=(jax.ShapeDtypeStruct((2, NP, D), _f32),
                  jax.ShapeDtypeStruct((2, NP, 2), _f32)),
        mesh=mesh,
        compiler_params=pltpu.CompilerParams(use_tc_tiling_on_sc=False),
        scratch_types=[
            pltpu.VMEM((CHUNKS, B), jnp.int32),
            pltpu.VMEM((CHUNKS, B), jnp.int32),
            pltpu.VMEM((2, B, D), _f32),
            pltpu.VMEM((2, B, 2), _f32),
            pltpu.VMEM_SHARED((NP, D), _f32),
            pltpu.VMEM_SHARED((NP, 2), _f32),
            pltpu.SemaphoreType.DMA,
            pltpu.SemaphoreType.DMA,
            pltpu.SemaphoreType.DMA,
            pltpu.SemaphoreType.DMA,
        ])


def kernel(features, edge_indices, edge_weights, W0, b0, A0, ab0,
           W1, b1, A1, ab1, Wp, bp, Wc1, bc1, Wc2, bc2):
    del edge_weights, ab0, ab1, Wp, bp
    edge_index = edge_indices[0]
    pad = N + jnp.arange(EP - E, dtype=jnp.int32) % (NP - N)
    src3 = jnp.concatenate([edge_index[0], pad]).reshape(NW, CHUNKS, B)
    dst3 = jnp.concatenate([edge_index[1], pad]).reshape(NW, CHUNKS, B)
    zg = jnp.zeros((STRIPE, D), _f32)
    ze = jnp.zeros((STRIPE, 2), _f32)

    sc = _sc_segsum_call()
    g0, e0, h0, f0 = _pre_call(features, W0, b0.reshape(1, D), A0)
    p, pt = sc(g0, e0, src3, dst3, zg, ze)
    x1 = _fin_call(p, pt, h0, f0)
    g1, e1, h1, f1 = _pre_call(x1, W1, b1.reshape(1, D), A1)
    q, qt = sc(g1, e1, src3, dst3, zg, ze)
    logits = _post_call(q, qt, h1, f1,
                        Wc1, bc1.reshape(1, HD), Wc2, bc2.reshape(1, 2))
    return logits
